# baseline (speedup 1.0000x reference)
import jax
import jax.numpy as jnp
from jax.experimental import pallas as pl


def _gcn_conv(x, edge_index, W, b):
    num_nodes = x.shape[0]
    src = edge_index[0]
    dst = edge_index[1]
    loop = jnp.arange(num_nodes, dtype=src.dtype)
    src = jnp.concatenate([src, loop])
    dst = jnp.concatenate([dst, loop])
    deg = jnp.zeros((num_nodes,), dtype=x.dtype).at[dst].add(1.0)
    deg_inv_sqrt = jnp.where(deg > 0, deg ** -0.5, 0.0)
    norm = deg_inv_sqrt[src] * deg_inv_sqrt[dst]
    h = x @ W
    msg = h[src] * norm[:, None]
    out = jnp.zeros((num_nodes, h.shape[1]), dtype=x.dtype).at[dst].add(msg)
    return out + b


def _identity_kernel(x_ref, o_ref):
    o_ref[...] = x_ref[...]


def kernel(x, edge_index, W1, b1, W2, b2, Wfc, bfc, Wfc2, bfc2):
    h = jax.nn.relu(_gcn_conv(x, edge_index, W1, b1))
    h = jax.nn.relu(_gcn_conv(h, edge_index, W2, b2))
    h = h.reshape(-1, 16 * 111)
    h = jax.nn.relu(h @ Wfc + bfc)
    h = h @ Wfc2 + bfc2
    h = jax.nn.sigmoid(h)
    return pl.pallas_call(
        _identity_kernel,
        out_shape=jax.ShapeDtypeStruct(h.shape, h.dtype),
    )(h)


# speedup vs baseline: 36.9137x; 36.9137x over previous
import functools

import jax
import jax.numpy as jnp
from jax import lax
from jax.experimental import pallas as pl
from jax.experimental.pallas import tpu as pltpu
from jax.experimental.pallas import tpu_sc as plsc

N = 33300
NP = 33408
D_IN = 128
F1 = 32
F2 = 16
NUM_NODES = 111
E = 532800
BLK = 128
SBLK = 4
NSB = 33
NT = 32
EP = NT * NSB * SBLK * BLK
EBLKS = EP // BLK
RPT = NP // 16
ZR = RPT // 8

_mesh = plsc.VectorSubcoreMesh(core_axis_name="c", subcore_axis_name="s")


def _zero16():
    return jnp.zeros((16,), jnp.float32)


def _fill_zeros(zb, width):
    def body(i, _):
        for k in range(width // 16):
            zb[i, pl.ds(k * 16, 16)] = _zero16()
        return 0
    lax.fori_loop(0, zb.shape[0], body, 0)


def _deg_body(dst2d, degp, acc, zb, ones, idx, sem):
    c = lax.axis_index("c")
    s = lax.axis_index("s")
    _fill_zeros(zb, F2)

    def fill_ones(i, _):
        ones[i, pl.ds(0, 16)] = _zero16() + 1.0
        return 0
    lax.fori_loop(0, BLK, fill_ones, 0)

    for b in range(8):
        pltpu.sync_copy(zb, acc.at[pl.ds(s * RPT + b * ZR, ZR)])
    plsc.subcore_barrier()

    wid = c * 16 + s
    blk0 = wid * (NSB * SBLK)

    def body(i, _):
        pltpu.sync_copy(dst2d.at[pl.ds(blk0 + i * SBLK, SBLK)], idx)
        for j in range(SBLK):
            pltpu.sync_copy(ones, acc.at[idx.at[j]], add=True)
        return 0
    lax.fori_loop(0, NSB, body, 0)
    plsc.subcore_barrier()
    pltpu.sync_copy(acc.at[pl.ds(s * RPT, RPT)], degp.at[c, pl.ds(s * RPT, RPT)])


_sc_params = pltpu.CompilerParams(use_tc_tiling_on_sc=False)

_deg_call = functools.partial(
    pl.kernel,
    mesh=_mesh,
    compiler_params=_sc_params,
    out_type=jax.ShapeDtypeStruct((2, NP, F2), jnp.float32),
    scratch_types=[
        pltpu.VMEM_SHARED((NP, F2), jnp.float32),
        pltpu.VMEM((ZR, F2), jnp.float32),
        pltpu.VMEM((BLK, F2), jnp.float32),
        pltpu.VMEM((SBLK, BLK), jnp.int32),
        pltpu.SemaphoreType.DMA,
    ],
)(_deg_body)


def _make_agg(F):
    def _agg_body(g, src2d, dst2d, aggp, acc, zb, sidx, didx, rows, sem):
        c = lax.axis_index("c")
        s = lax.axis_index("s")
        _fill_zeros(zb, F)
        for b in range(8):
            pltpu.sync_copy(zb, acc.at[pl.ds(s * RPT + b * ZR, ZR)])
        plsc.subcore_barrier()

        wid = c * 16 + s
        blk0 = wid * (NSB * SBLK)

        def body(i, _):
            pltpu.sync_copy(src2d.at[pl.ds(blk0 + i * SBLK, SBLK)], sidx)
            pltpu.sync_copy(dst2d.at[pl.ds(blk0 + i * SBLK, SBLK)], didx)
            for j in range(SBLK):
                pltpu.async_copy(g.at[sidx.at[j]], rows, sem).wait()
                pltpu.sync_copy(rows, acc.at[didx.at[j]], add=True)
            return 0
        lax.fori_loop(0, NSB, body, 0)
        plsc.subcore_barrier()
        pltpu.sync_copy(acc.at[pl.ds(s * RPT, RPT)],
                        aggp.at[c, pl.ds(s * RPT, RPT)])

    return functools.partial(
        pl.kernel,
        mesh=_mesh,
        compiler_params=_sc_params,
        out_type=jax.ShapeDtypeStruct((2, NP, F), jnp.float32),
        scratch_types=[
            pltpu.VMEM_SHARED((NP, F), jnp.float32),
            pltpu.VMEM((ZR, F), jnp.float32),
            pltpu.VMEM((SBLK, BLK), jnp.int32),
            pltpu.VMEM((SBLK, BLK), jnp.int32),
            pltpu.VMEM((BLK, F), jnp.float32),
            pltpu.SemaphoreType.DMA,
        ],
    )(_agg_body)


_agg1_call = _make_agg(F1)
_agg2_call = _make_agg(F2)

RB = 2088


def _pre1_body(xr, w1r, dpr, g1r, disr):
    deg = dpr[0, :, 0:1] + dpr[1, :, 0:1] + 1.0
    dis = lax.rsqrt(deg)
    h = jnp.dot(xr[...], w1r[...], preferred_element_type=jnp.float32)
    g1r[...] = dis * h
    disr[...] = dis


def _pre1(x_pad, W1, degp):
    return pl.pallas_call(
        _pre1_body,
        grid=(16,),
        in_specs=[
            pl.BlockSpec((RB, D_IN), lambda i: (i, 0)),
            pl.BlockSpec((D_IN, F1), lambda i: (0, 0)),
            pl.BlockSpec((2, RB, F2), lambda i: (0, i, 0)),
        ],
        out_specs=[
            pl.BlockSpec((RB, F1), lambda i: (i, 0)),
            pl.BlockSpec((RB, 1), lambda i: (i, 0)),
        ],
        out_shape=[
            jax.ShapeDtypeStruct((NP, F1), jnp.float32),
            jax.ShapeDtypeStruct((NP, 1), jnp.float32),
        ],
    )(x_pad, W1, degp)


def _mid_body(apr, g1r, disr, b1r, w2r, g2r):
    dis = disr[...]
    out1 = jnp.maximum(dis * (apr[0] + apr[1] + g1r[...]) + b1r[...], 0.0)
    h2 = jnp.dot(out1, w2r[...], preferred_element_type=jnp.float32)
    g2r[...] = dis * h2


def _mid(aggp1, g1, dis, b1, W2):
    return pl.pallas_call(
        _mid_body,
        grid=(16,),
        in_specs=[
            pl.BlockSpec((2, RB, F1), lambda i: (0, i, 0)),
            pl.BlockSpec((RB, F1), lambda i: (i, 0)),
            pl.BlockSpec((RB, 1), lambda i: (i, 0)),
            pl.BlockSpec((1, F1), lambda i: (0, 0)),
            pl.BlockSpec((F1, F2), lambda i: (0, 0)),
        ],
        out_specs=pl.BlockSpec((RB, F2), lambda i: (i, 0)),
        out_shape=jax.ShapeDtypeStruct((NP, F2), jnp.float32),
    )(aggp1, g1, dis, b1, W2)


def _post2_body(apr, g2r, disr, b2r, outr):
    dis = disr[...]
    outr[...] = jnp.maximum(dis * (apr[0] + apr[1] + g2r[...]) + b2r[...], 0.0)


def _post2(aggp2, g2, dis, b2):
    return pl.pallas_call(
        _post2_body,
        grid=(16,),
        in_specs=[
            pl.BlockSpec((2, RB, F2), lambda i: (0, i, 0)),
            pl.BlockSpec((RB, F2), lambda i: (i, 0)),
            pl.BlockSpec((RB, 1), lambda i: (i, 0)),
            pl.BlockSpec((1, F2), lambda i: (0, 0)),
        ],
        out_specs=pl.BlockSpec((RB, F2), lambda i: (i, 0)),
        out_shape=jax.ShapeDtypeStruct((N, F2), jnp.float32),
    )(aggp2, g2, dis, b2)


def _head_body(hr, wfcr, bfcr, wfc2r, bfc2r, outr):
    t = jnp.dot(hr[...], wfcr[...], preferred_element_type=jnp.float32)
    t = jnp.maximum(t + bfcr[...], 0.0)
    y = jnp.dot(t, wfc2r[...], preferred_element_type=jnp.float32)
    y = y + bfc2r[...]
    outr[...] = 1.0 / (1.0 + jnp.exp(-y))


def _head(hflat, Wfc, bfc, Wfc2, bfc2):
    return pl.pallas_call(
        _head_body,
        out_shape=jax.ShapeDtypeStruct((N // NUM_NODES, 1), jnp.float32),
    )(hflat, Wfc, bfc, Wfc2, bfc2)


def kernel(x, edge_index, W1, b1, W2, b2, Wfc, bfc, Wfc2, bfc2):
    ei = edge_index.astype(jnp.int32)
    pad = jnp.full((EP - E,), N, dtype=jnp.int32)
    src2d = jnp.concatenate([ei[0], pad]).reshape(EBLKS, BLK)
    dst2d = jnp.concatenate([ei[1], pad]).reshape(EBLKS, BLK)
    x_pad = jnp.pad(x, ((0, NP - N), (0, 0)))

    degp = _deg_call(dst2d)
    g1, dis = _pre1(x_pad, W1, degp)
    aggp1 = _agg1_call(g1, src2d, dst2d)
    g2 = _mid(aggp1, g1, dis, b1.reshape(1, F1), W2)
    aggp2 = _agg2_call(g2, src2d, dst2d)
    out2 = _post2(aggp2, g2, dis, b2.reshape(1, F2))
    hflat = out2.reshape(N // NUM_NODES, F2 * NUM_NODES)
    return _head(hflat, Wfc, bfc.reshape(1, 64), Wfc2, bfc2.reshape(1, 1))


# speedup vs baseline: 55.2412x; 1.4965x over previous
import functools

import jax
import jax.numpy as jnp
from jax import lax
from jax.experimental import pallas as pl
from jax.experimental.pallas import tpu as pltpu
from jax.experimental.pallas import tpu_sc as plsc

N = 33300
NP = 33408
D_IN = 128
F1 = 32
F2 = 16
NUM_NODES = 111
E = 532800
BLK = 128
NBLK = 132
NT = 32
EP = NT * NBLK * BLK
EBLKS = EP // BLK
RPT = NP // 16
RING = 6
LOOK = 3

_mesh = plsc.VectorSubcoreMesh(core_axis_name="c", subcore_axis_name="s")
_sc_params = pltpu.CompilerParams(use_tc_tiling_on_sc=False)


def _zero16():
    return jnp.zeros((16,), jnp.float32)


def _fill_zeros(zb, width):
    def body(i, _):
        for k in range(width // 16):
            zb[i, pl.ds(k * 16, 16)] = _zero16()
        return 0
    lax.fori_loop(0, zb.shape[0], body, 0)


def _zero_acc_slice(zrow, acc, s, sem):
    _fill_zeros(zrow, zrow.shape[1])
    for j in range(16):
        pltpu.async_copy(zrow, acc.at[pl.ds(s * RPT + j * BLK, BLK)], sem)
    pltpu.async_copy(zrow.at[pl.ds(0, RPT - 16 * BLK)],
                     acc.at[pl.ds(s * RPT + 16 * BLK, RPT - 16 * BLK)], sem)
    for j in range(16):
        pltpu.make_async_copy(zrow, acc.at[pl.ds(s * RPT + j * BLK, BLK)],
                              sem).wait()
    pltpu.make_async_copy(zrow.at[pl.ds(0, RPT - 16 * BLK)],
                          acc.at[pl.ds(s * RPT + 16 * BLK, RPT - 16 * BLK)],
                          sem).wait()


def _deg_body(dst2d, degp, acc, zrow, ones, didx, ssem):
    c = lax.axis_index("c")
    s = lax.axis_index("s")

    def fill_ones(i, _):
        ones[i, pl.ds(0, 16)] = _zero16() + 1.0
        return 0
    lax.fori_loop(0, BLK, fill_ones, 0)

    wid = c * 16 + s
    pltpu.sync_copy(dst2d.at[pl.ds(wid * NBLK, NBLK)], didx)
    _zero_acc_slice(zrow, acc, s, ssem)
    plsc.subcore_barrier()

    def fire(b, _):
        pltpu.async_copy(ones, acc.at[didx.at[b]], ssem, add=True)
        return 0
    lax.fori_loop(0, NBLK, fire, 0)

    def drain(b, _):
        pltpu.make_async_copy(ones, acc.at[didx.at[b]], ssem).wait()
        return 0
    lax.fori_loop(0, NBLK, drain, 0)
    plsc.subcore_barrier()
    pltpu.sync_copy(acc.at[pl.ds(s * RPT, RPT)], degp.at[c, pl.ds(s * RPT, RPT)])


_deg_call = functools.partial(
    pl.kernel,
    mesh=_mesh,
    compiler_params=_sc_params,
    out_type=jax.ShapeDtypeStruct((2, NP, F2), jnp.float32),
    scratch_types=[
        pltpu.VMEM_SHARED((NP, F2), jnp.float32),
        pltpu.VMEM((BLK, F2), jnp.float32),
        pltpu.VMEM((BLK, F2), jnp.float32),
        pltpu.VMEM((NBLK, BLK), jnp.int32),
        pltpu.SemaphoreType.DMA,
    ],
)(_deg_body)


def _make_agg(F):
    def _agg_body(g, src2d, dst2d, aggp, acc, sidx, didx,
                  r0, r1, r2, r3, r4, r5, gsem, ssem):
        rows = [r0, r1, r2, r3, r4, r5]
        c = lax.axis_index("c")
        s = lax.axis_index("s")
        wid = c * 16 + s
        pltpu.sync_copy(src2d.at[pl.ds(wid * NBLK, NBLK)], sidx)
        pltpu.sync_copy(dst2d.at[pl.ds(wid * NBLK, NBLK)], didx)
        _zero_acc_slice(r0, acc, s, ssem)
        plsc.subcore_barrier()

        for k in range(LOOK):
            pltpu.async_copy(g.at[sidx.at[k]], rows[k], gsem)

        def it(i, _):
            for k in range(RING):
                b = i * RING + k
                nslot = (k + LOOK) % RING

                @pl.when(b >= LOOK)
                def _():
                    pltpu.make_async_copy(
                        rows[nslot], acc.at[didx.at[b - LOOK]], ssem).wait()

                @pl.when(b + LOOK < NBLK)
                def _():
                    pltpu.async_copy(g.at[sidx.at[b + LOOK]], rows[nslot], gsem)

                pltpu.make_async_copy(g.at[sidx.at[b]], rows[k], gsem).wait()
                pltpu.async_copy(rows[k], acc.at[didx.at[b]], ssem, add=True)
            return 0
        lax.fori_loop(0, NBLK // RING, it, 0)

        for j in range(LOOK):
            b = NBLK - LOOK + j
            pltpu.make_async_copy(rows[b % RING], acc.at[didx.at[b]], ssem).wait()
        plsc.subcore_barrier()
        pltpu.sync_copy(acc.at[pl.ds(s * RPT, RPT)],
                        aggp.at[c, pl.ds(s * RPT, RPT)])

    return functools.partial(
        pl.kernel,
        mesh=_mesh,
        compiler_params=_sc_params,
        out_type=jax.ShapeDtypeStruct((2, NP, F), jnp.float32),
        scratch_types=[
            pltpu.VMEM_SHARED((NP, F), jnp.float32),
            pltpu.VMEM((NBLK, BLK), jnp.int32),
            pltpu.VMEM((NBLK, BLK), jnp.int32),
        ] + [pltpu.VMEM((BLK, F), jnp.float32)] * RING + [
            pltpu.SemaphoreType.DMA,
            pltpu.SemaphoreType.DMA,
        ],
    )(_agg_body)


_agg1_call = _make_agg(F1)
_agg2_call = _make_agg(F2)

RB = 2088


def _pre1_body(xr, w1r, dpr, g1r, disr):
    deg = dpr[0, :, 0:1] + dpr[1, :, 0:1] + 1.0
    dis = lax.rsqrt(deg)
    h = jnp.dot(xr[...], w1r[...], preferred_element_type=jnp.float32)
    g1r[...] = dis * h
    disr[...] = dis


def _pre1(x_pad, W1, degp):
    return pl.pallas_call(
        _pre1_body,
        grid=(16,),
        in_specs=[
            pl.BlockSpec((RB, D_IN), lambda i: (i, 0)),
            pl.BlockSpec((D_IN, F1), lambda i: (0, 0)),
            pl.BlockSpec((2, RB, F2), lambda i: (0, i, 0)),
        ],
        out_specs=[
            pl.BlockSpec((RB, F1), lambda i: (i, 0)),
            pl.BlockSpec((RB, 1), lambda i: (i, 0)),
        ],
        out_shape=[
            jax.ShapeDtypeStruct((NP, F1), jnp.float32),
            jax.ShapeDtypeStruct((NP, 1), jnp.float32),
        ],
    )(x_pad, W1, degp)


def _mid_body(apr, g1r, disr, b1r, w2r, g2r):
    dis = disr[...]
    out1 = jnp.maximum(dis * (apr[0] + apr[1] + g1r[...]) + b1r[...], 0.0)
    h2 = jnp.dot(out1, w2r[...], preferred_element_type=jnp.float32)
    g2r[...] = dis * h2


def _mid(aggp1, g1, dis, b1, W2):
    return pl.pallas_call(
        _mid_body,
        grid=(16,),
        in_specs=[
            pl.BlockSpec((2, RB, F1), lambda i: (0, i, 0)),
            pl.BlockSpec((RB, F1), lambda i: (i, 0)),
            pl.BlockSpec((RB, 1), lambda i: (i, 0)),
            pl.BlockSpec((1, F1), lambda i: (0, 0)),
            pl.BlockSpec((F1, F2), lambda i: (0, 0)),
        ],
        out_specs=pl.BlockSpec((RB, F2), lambda i: (i, 0)),
        out_shape=jax.ShapeDtypeStruct((NP, F2), jnp.float32),
    )(aggp1, g1, dis, b1, W2)


def _post2_body(apr, g2r, disr, b2r, outr):
    dis = disr[...]
    outr[...] = jnp.maximum(dis * (apr[0] + apr[1] + g2r[...]) + b2r[...], 0.0)


def _post2(aggp2, g2, dis, b2):
    return pl.pallas_call(
        _post2_body,
        grid=(16,),
        in_specs=[
            pl.BlockSpec((2, RB, F2), lambda i: (0, i, 0)),
            pl.BlockSpec((RB, F2), lambda i: (i, 0)),
            pl.BlockSpec((RB, 1), lambda i: (i, 0)),
            pl.BlockSpec((1, F2), lambda i: (0, 0)),
        ],
        out_specs=pl.BlockSpec((RB, F2), lambda i: (i, 0)),
        out_shape=jax.ShapeDtypeStruct((N, F2), jnp.float32),
    )(aggp2, g2, dis, b2)


def _head_body(hr, wfcr, bfcr, wfc2r, bfc2r, outr):
    t = jnp.dot(hr[...], wfcr[...], preferred_element_type=jnp.float32)
    t = jnp.maximum(t + bfcr[...], 0.0)
    y = jnp.dot(t, wfc2r[...], preferred_element_type=jnp.float32)
    y = y + bfc2r[...]
    outr[...] = 1.0 / (1.0 + jnp.exp(-y))


def _head(hflat, Wfc, bfc, Wfc2, bfc2):
    return pl.pallas_call(
        _head_body,
        out_shape=jax.ShapeDtypeStruct((N // NUM_NODES, 1), jnp.float32),
    )(hflat, Wfc, bfc, Wfc2, bfc2)


def kernel(x, edge_index, W1, b1, W2, b2, Wfc, bfc, Wfc2, bfc2):
    ei = edge_index.astype(jnp.int32)
    pad = jnp.full((EP - E,), N, dtype=jnp.int32)
    src2d = jnp.concatenate([ei[0], pad]).reshape(EBLKS, BLK)
    dst2d = jnp.concatenate([ei[1], pad]).reshape(EBLKS, BLK)
    x_pad = jnp.pad(x, ((0, NP - N), (0, 0)))

    degp = _deg_call(dst2d)
    g1, dis = _pre1(x_pad, W1, degp)
    aggp1 = _agg1_call(g1, src2d, dst2d)
    g2 = _mid(aggp1, g1, dis, b1.reshape(1, F1), W2)
    aggp2 = _agg2_call(g2, src2d, dst2d)
    out2 = _post2(aggp2, g2, dis, b2.reshape(1, F2))
    hflat = out2.reshape(N // NUM_NODES, F2 * NUM_NODES)
    return _head(hflat, Wfc, bfc.reshape(1, 64), Wfc2, bfc2.reshape(1, 1))


# speedup vs baseline: 71.7479x; 1.2988x over previous
import functools

import jax
import jax.numpy as jnp
from jax import lax
from jax.experimental import pallas as pl
from jax.experimental.pallas import tpu as pltpu
from jax.experimental.pallas import tpu_sc as plsc

N = 33300
NP = 33408
NP8 = NP // 8
D_IN = 128
F1 = 32
F2 = 16
NUM_NODES = 111
E = 532800
BLK = 128
NBLK = 132
NT = 32
EP = NT * NBLK * BLK
EBLKS = EP // BLK
RPT = NP // 16
RING = 6
LOOK = 3

_mesh = plsc.VectorSubcoreMesh(core_axis_name="c", subcore_axis_name="s")
_sc_params = pltpu.CompilerParams(use_tc_tiling_on_sc=False)


def _zero16():
    return jnp.zeros((16,), jnp.float32)


def _fill_zeros(zb, width):
    def body(i, _):
        for k in range(width // 16):
            zb[i, pl.ds(k * 16, 16)] = _zero16()
        return 0
    lax.fori_loop(0, zb.shape[0], body, 0)


def _zero_acc_slice(zrow, acc, s, sem):
    _fill_zeros(zrow, zrow.shape[1])
    for j in range(16):
        pltpu.async_copy(zrow, acc.at[pl.ds(s * RPT + j * BLK, BLK)], sem)
    pltpu.async_copy(zrow.at[pl.ds(0, RPT - 16 * BLK)],
                     acc.at[pl.ds(s * RPT + 16 * BLK, RPT - 16 * BLK)], sem)
    for j in range(16):
        pltpu.make_async_copy(zrow, acc.at[pl.ds(s * RPT + j * BLK, BLK)],
                              sem).wait()
    pltpu.make_async_copy(zrow.at[pl.ds(0, RPT - 16 * BLK)],
                          acc.at[pl.ds(s * RPT + 16 * BLK, RPT - 16 * BLK)],
                          sem).wait()


def _deg_body(e2d, degp, acc, zrow, ones, didx, ssem):
    c = lax.axis_index("c")
    s = lax.axis_index("s")

    def fill_ones(i, _):
        ones[i, pl.ds(0, 16)] = _zero16() + 1.0
        return 0
    lax.fori_loop(0, BLK, fill_ones, 0)

    wid = c * 16 + s
    pltpu.sync_copy(e2d.at[1, pl.ds(wid * NBLK, NBLK)], didx)
    _zero_acc_slice(zrow, acc, s, ssem)
    plsc.subcore_barrier()

    def fire(b, _):
        pltpu.async_copy(ones, acc.at[didx.at[b]], ssem, add=True)
        return 0
    lax.fori_loop(0, NBLK, fire, 0)

    def drain(b, _):
        pltpu.make_async_copy(ones, acc.at[didx.at[b]], ssem).wait()
        return 0
    lax.fori_loop(0, NBLK, drain, 0)
    plsc.subcore_barrier()
    pltpu.sync_copy(acc.at[pl.ds(s * RPT, RPT)], degp.at[c, pl.ds(s * RPT, RPT)])


_deg_call = functools.partial(
    pl.kernel,
    mesh=_mesh,
    compiler_params=_sc_params,
    out_type=jax.ShapeDtypeStruct((2, NP, F2), jnp.float32),
    scratch_types=[
        pltpu.VMEM_SHARED((NP, F2), jnp.float32),
        pltpu.VMEM((BLK, F2), jnp.float32),
        pltpu.VMEM((BLK, F2), jnp.float32),
        pltpu.VMEM((NBLK, BLK), jnp.int32),
        pltpu.SemaphoreType.DMA,
    ],
)(_deg_body)


def _make_agg(F):
    def _agg_body(g, e2d, aggp, acc, sidx, didx,
                  r0, r1, r2, r3, r4, r5, gsem, ssem):
        rows = [r0, r1, r2, r3, r4, r5]
        c = lax.axis_index("c")
        s = lax.axis_index("s")
        wid = c * 16 + s
        pltpu.sync_copy(e2d.at[0, pl.ds(wid * NBLK, NBLK)], sidx)
        pltpu.sync_copy(e2d.at[1, pl.ds(wid * NBLK, NBLK)], didx)
        _zero_acc_slice(r0, acc, s, ssem)
        plsc.subcore_barrier()

        for k in range(LOOK):
            pltpu.async_copy(g.at[sidx.at[k]], rows[k], gsem)

        def it(i, _):
            for k in range(RING):
                b = i * RING + k
                nslot = (k + LOOK) % RING

                @pl.when(b >= LOOK)
                def _():
                    pltpu.make_async_copy(
                        rows[nslot], acc.at[didx.at[b - LOOK]], ssem).wait()

                @pl.when(b + LOOK < NBLK)
                def _():
                    pltpu.async_copy(g.at[sidx.at[b + LOOK]], rows[nslot], gsem)

                pltpu.make_async_copy(g.at[sidx.at[b]], rows[k], gsem).wait()
                pltpu.async_copy(rows[k], acc.at[didx.at[b]], ssem, add=True)
            return 0
        lax.fori_loop(0, NBLK // RING, it, 0)

        for j in range(LOOK):
            b = NBLK - LOOK + j
            pltpu.make_async_copy(rows[b % RING], acc.at[didx.at[b]], ssem).wait()
        plsc.subcore_barrier()
        pltpu.sync_copy(acc.at[pl.ds(s * RPT, RPT)],
                        aggp.at[c, pl.ds(s * RPT, RPT)])

    return functools.partial(
        pl.kernel,
        mesh=_mesh,
        compiler_params=_sc_params,
        out_type=jax.ShapeDtypeStruct((2, NP, F), jnp.float32),
        scratch_types=[
            pltpu.VMEM_SHARED((NP, F), jnp.float32),
            pltpu.VMEM((NBLK, BLK), jnp.int32),
            pltpu.VMEM((NBLK, BLK), jnp.int32),
        ] + [pltpu.VMEM((BLK, F), jnp.float32)] * RING + [
            pltpu.SemaphoreType.DMA,
            pltpu.SemaphoreType.DMA,
        ],
    )(_agg_body)


_agg1_call = _make_agg(F1)
_agg2_call = _make_agg(F2)

RB8 = 2088


def _h1_body(x8r, w1br, h1r):
    h1r[...] = jnp.dot(x8r[...], w1br[...], preferred_element_type=jnp.float32)


def _h1(x8, W1b):
    return pl.pallas_call(
        _h1_body,
        grid=(2,),
        in_specs=[
            pl.BlockSpec((RB8, 8 * D_IN), lambda i: (i, 0)),
            pl.BlockSpec((8 * D_IN, 8 * F1), lambda i: (0, 0)),
        ],
        out_specs=pl.BlockSpec((RB8, 8 * F1), lambda i: (i, 0)),
        out_shape=jax.ShapeDtypeStruct((NP8, 8 * F1), jnp.float32),
    )(x8, W1b)


def _dis16(dpr):
    return lax.rsqrt(dpr[0] + dpr[1] + 1.0)


def _pre1b_body(h1r, dpr, q8r, g1r):
    rep32 = jnp.dot(_dis16(dpr), q8r[...], preferred_element_type=jnp.float32)
    g1r[...] = rep32 * h1r[...]


def _pre1b(h1p, degv, Q8):
    return pl.pallas_call(
        _pre1b_body,
        grid=(2,),
        in_specs=[
            pl.BlockSpec((RB8, 8 * F1), lambda i: (i, 0)),
            pl.BlockSpec((2, RB8, 8 * F2), lambda i: (0, i, 0)),
            pl.BlockSpec((8 * F2, 8 * F1), lambda i: (0, 0)),
        ],
        out_specs=pl.BlockSpec((RB8, 8 * F1), lambda i: (i, 0)),
        out_shape=jax.ShapeDtypeStruct((NP8, 8 * F1), jnp.float32),
    )(h1p, degv, Q8)


def _mid_body(apr, g1r, dpr, q8r, b1r, w2br, g2r):
    dis16 = _dis16(dpr)
    rep32 = jnp.dot(dis16, q8r[...], preferred_element_type=jnp.float32)
    out1 = jnp.maximum(rep32 * (apr[0] + apr[1] + g1r[...]) + b1r[...], 0.0)
    g2r[...] = dis16 * jnp.dot(out1, w2br[...],
                               preferred_element_type=jnp.float32)


def _mid(aggv1, g1p, degv, Q8, b1t, W2b):
    return pl.pallas_call(
        _mid_body,
        grid=(2,),
        in_specs=[
            pl.BlockSpec((2, RB8, 8 * F1), lambda i: (0, i, 0)),
            pl.BlockSpec((RB8, 8 * F1), lambda i: (i, 0)),
            pl.BlockSpec((2, RB8, 8 * F2), lambda i: (0, i, 0)),
            pl.BlockSpec((8 * F2, 8 * F1), lambda i: (0, 0)),
            pl.BlockSpec((1, 8 * F1), lambda i: (0, 0)),
            pl.BlockSpec((8 * F1, 8 * F2), lambda i: (0, 0)),
        ],
        out_specs=pl.BlockSpec((RB8, 8 * F2), lambda i: (i, 0)),
        out_shape=jax.ShapeDtypeStruct((NP8, 8 * F2), jnp.float32),
    )(aggv1, g1p, degv, Q8, b1t, W2b)


def _post2_body(apr, g2r, dpr, b2r, outr):
    dis16 = _dis16(dpr)
    outr[...] = jnp.maximum(
        dis16 * (apr[0] + apr[1] + g2r[...]) + b2r[...], 0.0)


def _post2(aggv2, g2p, degv, b2t):
    return pl.pallas_call(
        _post2_body,
        grid=(2,),
        in_specs=[
            pl.BlockSpec((2, RB8, 8 * F2), lambda i: (0, i, 0)),
            pl.BlockSpec((RB8, 8 * F2), lambda i: (i, 0)),
            pl.BlockSpec((2, RB8, 8 * F2), lambda i: (0, i, 0)),
            pl.BlockSpec((1, 8 * F2), lambda i: (0, 0)),
        ],
        out_specs=pl.BlockSpec((RB8, 8 * F2), lambda i: (i, 0)),
        out_shape=jax.ShapeDtypeStruct((NP8, 8 * F2), jnp.float32),
    )(aggv2, g2p, degv, b2t)


def _head_body(hr, wfcr, bfcr, wfc2r, bfc2r, outr):
    t = jnp.dot(hr[...], wfcr[...], preferred_element_type=jnp.float32)
    t = jnp.maximum(t + bfcr[...], 0.0)
    y = jnp.dot(t, wfc2r[...], preferred_element_type=jnp.float32)
    y = y + bfc2r[...]
    outr[...] = 1.0 / (1.0 + jnp.exp(-y))


def _head(hflat, Wfc, bfc, Wfc2, bfc2):
    return pl.pallas_call(
        _head_body,
        out_shape=jax.ShapeDtypeStruct((N // NUM_NODES, 1), jnp.float32),
    )(hflat, Wfc, bfc, Wfc2, bfc2)


def kernel(x, edge_index, W1, b1, W2, b2, Wfc, bfc, Wfc2, bfc2):
    f32 = jnp.float32
    ei = edge_index.astype(jnp.int32)
    e2d = jnp.pad(ei, ((0, 0), (0, EP - E)),
                  constant_values=N).reshape(2, EBLKS, BLK)
    x8 = jnp.pad(x, ((0, NP - N), (0, 0))).reshape(NP8, 8 * D_IN)

    eye16x2 = jnp.concatenate(
        [jnp.eye(16, dtype=f32), jnp.eye(16, dtype=f32)], axis=1)
    Q8 = jnp.kron(jnp.eye(8, dtype=f32), eye16x2)
    W1b = jnp.kron(jnp.eye(8, dtype=f32), W1)
    W2b = jnp.kron(jnp.eye(8, dtype=f32), W2)
    b1t = jnp.tile(b1, 8).reshape(1, 8 * F1)
    b2t = jnp.tile(b2, 8).reshape(1, 8 * F2)

    degp = _deg_call(e2d)
    degv = degp.reshape(2, NP8, 8 * F2)
    h1p = _h1(x8, W1b)
    g1p = _pre1b(h1p, degv, Q8)

    aggp1 = _agg1_call(g1p.reshape(NP, F1), e2d)
    g2p = _mid(aggp1.reshape(2, NP8, 8 * F1), g1p, degv, Q8, b1t, W2b)

    aggp2 = _agg2_call(g2p.reshape(NP, F2), e2d)
    out2p = _post2(aggp2.reshape(2, NP8, 8 * F2), g2p, degv, b2t)

    hflat = out2p.reshape(-1)[: N * F2].reshape(N // NUM_NODES,
                                                F2 * NUM_NODES)
    return _head(hflat, Wfc, bfc.reshape(1, 64), Wfc2, bfc2.reshape(1, 1))


# speedup vs baseline: 77.3806x; 1.0785x over previous
import functools

import jax
import jax.numpy as jnp
from jax import lax
from jax.experimental import pallas as pl
from jax.experimental.pallas import tpu as pltpu
from jax.experimental.pallas import tpu_sc as plsc

N = 33300
NP = 33408
NP8 = NP // 8
D_IN = 128
F1 = 32
F2 = 16
NUM_NODES = 111
E = 532800
BLK = 128
NBLKT = 264
NT = 32
EP = 16 * NBLKT * BLK
EBLKS = EP // BLK
RPT = NP // 16
DEG_SPLIT = (152, 112)
AGG1_SPLIT = (184, 80)
AGG2_SPLIT = (162, 102)

_mesh = plsc.VectorSubcoreMesh(core_axis_name="c", subcore_axis_name="s")
_sc_params = pltpu.CompilerParams(use_tc_tiling_on_sc=False)


def _zero16():
    return jnp.zeros((16,), jnp.float32)


def _fill_zeros(zb, width):
    def body(i, _):
        for k in range(width // 16):
            zb[i, pl.ds(k * 16, 16)] = _zero16()
        return 0
    lax.fori_loop(0, zb.shape[0], body, 0)


def _zero_acc_slice(zrow, acc, s, sem):
    _fill_zeros(zrow, zrow.shape[1])
    for j in range(16):
        pltpu.async_copy(zrow, acc.at[pl.ds(s * RPT + j * BLK, BLK)], sem)
    pltpu.async_copy(zrow.at[pl.ds(0, RPT - 16 * BLK)],
                     acc.at[pl.ds(s * RPT + 16 * BLK, RPT - 16 * BLK)], sem)
    for j in range(16):
        pltpu.make_async_copy(zrow, acc.at[pl.ds(s * RPT + j * BLK, BLK)],
                              sem).wait()
    pltpu.make_async_copy(zrow.at[pl.ds(0, RPT - 16 * BLK)],
                          acc.at[pl.ds(s * RPT + 16 * BLK, RPT - 16 * BLK)],
                          sem).wait()


def _deg_run(e2d, acc, ones, didx, ssem, blk0, nblk):
    pltpu.sync_copy(e2d.at[1, pl.ds(blk0, nblk)], didx.at[pl.ds(0, nblk)])

    def fire(b, _):
        pltpu.async_copy(ones, acc.at[didx.at[b]], ssem, add=True)
        return 0
    lax.fori_loop(0, nblk, fire, 0)

    def drain(b, _):
        pltpu.make_async_copy(ones, acc.at[didx.at[b]], ssem).wait()
        return 0
    lax.fori_loop(0, nblk, drain, 0)


def _deg_body(e2d, degp, acc, zrow, ones, didx, ssem):
    c = lax.axis_index("c")
    s = lax.axis_index("s")
    nb0, nb1 = DEG_SPLIT

    def fill_ones(i, _):
        ones[i, pl.ds(0, 16)] = _zero16() + 1.0
        return 0
    lax.fori_loop(0, BLK, fill_ones, 0)
    _zero_acc_slice(zrow, acc, s, ssem)
    plsc.subcore_barrier()

    @pl.when(c == 0)
    def _():
        _deg_run(e2d, acc, ones, didx, ssem, s * nb0, nb0)

    @pl.when(c == 1)
    def _():
        _deg_run(e2d, acc, ones, didx, ssem, 16 * nb0 + s * nb1, nb1)

    plsc.subcore_barrier()
    pltpu.sync_copy(acc.at[pl.ds(s * RPT, RPT)], degp.at[c, pl.ds(s * RPT, RPT)])


_deg_call = functools.partial(
    pl.kernel,
    mesh=_mesh,
    compiler_params=_sc_params,
    out_type=jax.ShapeDtypeStruct((2, NP, F2), jnp.float32),
    scratch_types=[
        pltpu.VMEM_SHARED((NP, F2), jnp.float32),
        pltpu.VMEM((BLK, F2), jnp.float32),
        pltpu.VMEM((BLK, F2), jnp.float32),
        pltpu.VMEM((DEG_SPLIT[0], BLK), jnp.int32),
        pltpu.SemaphoreType.DMA,
    ],
)(_deg_body)


def _agg_run(g, e2d, acc, sidx, didx, rows, gsem, ssem, blk0, nblk, ring, look):
    pltpu.sync_copy(e2d.at[0, pl.ds(blk0, nblk)], sidx.at[pl.ds(0, nblk)])
    pltpu.sync_copy(e2d.at[1, pl.ds(blk0, nblk)], didx.at[pl.ds(0, nblk)])

    for k in range(look):
        pltpu.async_copy(g.at[sidx.at[k]], rows[k], gsem)

    def it(i, _):
        for k in range(ring):
            b = i * ring + k
            nslot = (k + look) % ring

            @pl.when(b >= look)
            def _():
                pltpu.make_async_copy(
                    rows[nslot], acc.at[didx.at[b - look]], ssem).wait()

            @pl.when(b + look < nblk)
            def _():
                pltpu.async_copy(g.at[sidx.at[b + look]], rows[nslot], gsem)

            pltpu.make_async_copy(g.at[sidx.at[b]], rows[k], gsem).wait()
            pltpu.async_copy(rows[k], acc.at[didx.at[b]], ssem, add=True)
        return 0
    lax.fori_loop(0, nblk // ring, it, 0)

    for j in range(look):
        b = nblk - look + j
        pltpu.make_async_copy(rows[b % ring], acc.at[didx.at[b]], ssem).wait()


def _make_agg(F, split, ring, look):
    nb0, nb1 = split

    def _agg_body(g, e2d, aggp, acc, sidx, didx, *rest):
        rows, (gsem, ssem) = list(rest[:ring]), rest[ring:]
        c = lax.axis_index("c")
        s = lax.axis_index("s")
        _zero_acc_slice(rows[0], acc, s, ssem)
        plsc.subcore_barrier()

        @pl.when(c == 0)
        def _():
            _agg_run(g, e2d, acc, sidx, didx, rows, gsem, ssem,
                     s * nb0, nb0, ring, look)

        @pl.when(c == 1)
        def _():
            _agg_run(g, e2d, acc, sidx, didx, rows, gsem, ssem,
                     16 * nb0 + s * nb1, nb1, ring, look)

        plsc.subcore_barrier()
        pltpu.sync_copy(acc.at[pl.ds(s * RPT, RPT)],
                        aggp.at[c, pl.ds(s * RPT, RPT)])

    return functools.partial(
        pl.kernel,
        mesh=_mesh,
        compiler_params=_sc_params,
        out_type=jax.ShapeDtypeStruct((2, NP, F), jnp.float32),
        scratch_types=[
            pltpu.VMEM_SHARED((NP, F), jnp.float32),
            pltpu.VMEM((nb0, BLK), jnp.int32),
            pltpu.VMEM((nb0, BLK), jnp.int32),
        ] + [pltpu.VMEM((BLK, F), jnp.float32)] * ring + [
            pltpu.SemaphoreType.DMA,
            pltpu.SemaphoreType.DMA,
        ],
    )(_agg_body)


_agg1_call = _make_agg(F1, AGG1_SPLIT, 4, 2)
_agg2_call = _make_agg(F2, AGG2_SPLIT, 6, 3)

RB8 = 2088


def _h1_body(x8r, w1br, h1r):
    h1r[...] = jnp.dot(x8r[...], w1br[...], preferred_element_type=jnp.float32)


def _h1(x8, W1b):
    return pl.pallas_call(
        _h1_body,
        grid=(2,),
        in_specs=[
            pl.BlockSpec((RB8, 8 * D_IN), lambda i: (i, 0)),
            pl.BlockSpec((8 * D_IN, 8 * F1), lambda i: (0, 0)),
        ],
        out_specs=pl.BlockSpec((RB8, 8 * F1), lambda i: (i, 0)),
        out_shape=jax.ShapeDtypeStruct((NP8, 8 * F1), jnp.float32),
    )(x8, W1b)


def _dis16(dpr):
    return lax.rsqrt(dpr[0] + dpr[1] + 1.0)


def _pre1b_body(h1r, dpr, q8r, g1r):
    rep32 = jnp.dot(_dis16(dpr), q8r[...], preferred_element_type=jnp.float32)
    g1r[...] = rep32 * h1r[...]


def _pre1b(h1p, degv, Q8):
    return pl.pallas_call(
        _pre1b_body,
        grid=(2,),
        in_specs=[
            pl.BlockSpec((RB8, 8 * F1), lambda i: (i, 0)),
            pl.BlockSpec((2, RB8, 8 * F2), lambda i: (0, i, 0)),
            pl.BlockSpec((8 * F2, 8 * F1), lambda i: (0, 0)),
        ],
        out_specs=pl.BlockSpec((RB8, 8 * F1), lambda i: (i, 0)),
        out_shape=jax.ShapeDtypeStruct((NP8, 8 * F1), jnp.float32),
    )(h1p, degv, Q8)


def _mid_body(apr, g1r, dpr, q8r, b1r, w2br, g2r):
    dis16 = _dis16(dpr)
    rep32 = jnp.dot(dis16, q8r[...], preferred_element_type=jnp.float32)
    out1 = jnp.maximum(rep32 * (apr[0] + apr[1] + g1r[...]) + b1r[...], 0.0)
    g2r[...] = dis16 * jnp.dot(out1, w2br[...],
                               preferred_element_type=jnp.float32)


def _mid(aggv1, g1p, degv, Q8, b1t, W2b):
    return pl.pallas_call(
        _mid_body,
        grid=(2,),
        in_specs=[
            pl.BlockSpec((2, RB8, 8 * F1), lambda i: (0, i, 0)),
            pl.BlockSpec((RB8, 8 * F1), lambda i: (i, 0)),
            pl.BlockSpec((2, RB8, 8 * F2), lambda i: (0, i, 0)),
            pl.BlockSpec((8 * F2, 8 * F1), lambda i: (0, 0)),
            pl.BlockSpec((1, 8 * F1), lambda i: (0, 0)),
            pl.BlockSpec((8 * F1, 8 * F2), lambda i: (0, 0)),
        ],
        out_specs=pl.BlockSpec((RB8, 8 * F2), lambda i: (i, 0)),
        out_shape=jax.ShapeDtypeStruct((NP8, 8 * F2), jnp.float32),
    )(aggv1, g1p, degv, Q8, b1t, W2b)


def _post2_body(apr, g2r, dpr, b2r, outr):
    dis16 = _dis16(dpr)
    outr[...] = jnp.maximum(
        dis16 * (apr[0] + apr[1] + g2r[...]) + b2r[...], 0.0)


def _post2(aggv2, g2p, degv, b2t):
    return pl.pallas_call(
        _post2_body,
        grid=(2,),
        in_specs=[
            pl.BlockSpec((2, RB8, 8 * F2), lambda i: (0, i, 0)),
            pl.BlockSpec((RB8, 8 * F2), lambda i: (i, 0)),
            pl.BlockSpec((2, RB8, 8 * F2), lambda i: (0, i, 0)),
            pl.BlockSpec((1, 8 * F2), lambda i: (0, 0)),
        ],
        out_specs=pl.BlockSpec((RB8, 8 * F2), lambda i: (i, 0)),
        out_shape=jax.ShapeDtypeStruct((NP8, 8 * F2), jnp.float32),
    )(aggv2, g2p, degv, b2t)


def _head_body(hr, wfcr, bfcr, wfc2r, bfc2r, outr):
    t = jnp.dot(hr[...], wfcr[...], preferred_element_type=jnp.float32)
    t = jnp.maximum(t + bfcr[...], 0.0)
    y = jnp.dot(t, wfc2r[...], preferred_element_type=jnp.float32)
    y = y + bfc2r[...]
    outr[...] = 1.0 / (1.0 + jnp.exp(-y))


def _head(hflat, Wfc, bfc, Wfc2, bfc2):
    return pl.pallas_call(
        _head_body,
        out_shape=jax.ShapeDtypeStruct((N // NUM_NODES, 1), jnp.float32),
    )(hflat, Wfc, bfc, Wfc2, bfc2)


def kernel(x, edge_index, W1, b1, W2, b2, Wfc, bfc, Wfc2, bfc2):
    f32 = jnp.float32
    ei = edge_index.astype(jnp.int32)
    e2d = jnp.pad(ei, ((0, 0), (0, EP - E)),
                  constant_values=N).reshape(2, EBLKS, BLK)
    x8 = jnp.pad(x, ((0, NP - N), (0, 0))).reshape(NP8, 8 * D_IN)

    eye16x2 = jnp.concatenate(
        [jnp.eye(16, dtype=f32), jnp.eye(16, dtype=f32)], axis=1)
    Q8 = jnp.kron(jnp.eye(8, dtype=f32), eye16x2)
    W1b = jnp.kron(jnp.eye(8, dtype=f32), W1)
    W2b = jnp.kron(jnp.eye(8, dtype=f32), W2)
    b1t = jnp.tile(b1, 8).reshape(1, 8 * F1)
    b2t = jnp.tile(b2, 8).reshape(1, 8 * F2)

    degp = _deg_call(e2d)
    degv = degp.reshape(2, NP8, 8 * F2)
    h1p = _h1(x8, W1b)
    g1p = _pre1b(h1p, degv, Q8)

    aggp1 = _agg1_call(g1p.reshape(NP, F1), e2d)
    g2p = _mid(aggp1.reshape(2, NP8, 8 * F1), g1p, degv, Q8, b1t, W2b)

    aggp2 = _agg2_call(g2p.reshape(NP, F2), e2d)
    out2p = _post2(aggp2.reshape(2, NP8, 8 * F2), g2p, degv, b2t)

    hflat = out2p.reshape(-1)[: N * F2].reshape(N // NUM_NODES,
                                                F2 * NUM_NODES)
    return _head(hflat, Wfc, bfc.reshape(1, 64), Wfc2, bfc2.reshape(1, 1))
